# baseline (speedup 1.0000x reference)
import functools

import jax
import jax.numpy as jnp
from jax import lax
from jax.experimental import pallas as pl
from jax.experimental.pallas import tpu as pltpu
from jax.experimental.pallas import tpu_sc as plsc

N_NODES = 10000
N_EDGES = 320000
DIM = 128
NC = 2
NS = 16
NW = NC * NS
L = 16
EPW = N_EDGES // NW
CHUNK = 80
NCHUNK = EPW // CHUNK
NF = DIM // L
NB = DIM // (2 * L)


def _pack_pair(a, b):
    ia = lax.bitcast_convert_type(a, jnp.int32)
    ib = lax.bitcast_convert_type(b, jnp.int32)
    lo = lax.shift_right_logical(ia + 0x8000, 16)
    hi = jnp.bitwise_and(ib + 0x8000, jnp.int32(-65536))
    return jnp.bitwise_or(hi, lo)


def _proj_body(x_ref, wal_ref, wah_ref, wbl_ref, wbh_ref, b1l_ref, b1h_ref,
               u_ref, v_ref):
    xv = x_ref[...]
    f32 = jnp.float32
    ul = jnp.dot(xv, wal_ref[...], preferred_element_type=f32)
    uh = jnp.dot(xv, wah_ref[...], preferred_element_type=f32)
    vl = jnp.dot(xv, wbl_ref[...], preferred_element_type=f32) + b1l_ref[...]
    vh = jnp.dot(xv, wbh_ref[...], preferred_element_type=f32) + b1h_ref[...]
    u_ref[...] = _pack_pair(ul, uh)
    v_ref[...] = _pack_pair(vl, vh)


def _node_projections(x, W1, b1):
    H = DIM // 2
    wa = W1[:, :DIM].T
    wb = W1[:, DIM:].T
    return pl.pallas_call(
        _proj_body,
        out_shape=[
            jax.ShapeDtypeStruct((N_NODES, H), jnp.int32),
            jax.ShapeDtypeStruct((N_NODES, H), jnp.int32),
        ],
    )(x, wa[:, :H], wa[:, H:], wb[:, :H], wb[:, H:],
      b1[:H].reshape(1, H), b1[H:].reshape(1, H))


def _edge_body(u_hbm, v_hbm, ei_hbm, w2_hbm, b2_hbm, out_hbm,
               idx_r, idx_c, urows, vrows, logv, w2v, b2v, sem):
    cid = lax.axis_index("c")
    sid = lax.axis_index("s")
    wid = sid * NC + cid
    base = wid * EPW

    pltpu.sync_copy(ei_hbm.at[0, wid], idx_r)
    pltpu.sync_copy(ei_hbm.at[1, wid], idx_c)
    pltpu.sync_copy(w2_hbm, w2v)
    pltpu.sync_copy(b2_hbm, b2v)
    b2reg = b2v[...]
    w2regs = [
        plsc.bitcast(w2v[pl.ds(b * L, L)], jnp.bfloat16) for b in range(NB)
    ]
    lane = lax.iota(jnp.int32, L)

    def issue(cc, s):
        pltpu.async_copy(u_hbm.at[idx_r.at[cc]], urows.at[s], sem.at[s])
        pltpu.async_copy(v_hbm.at[idx_c.at[cc]], vrows.at[s], sem.at[s])

    def wait(cc, s):
        pltpu.make_async_copy(u_hbm.at[idx_r.at[cc]], urows.at[s], sem.at[s]).wait()
        pltpu.make_async_copy(v_hbm.at[idx_c.at[cc]], vrows.at[s], sem.at[s]).wait()

    def compute(cc, s):
        ur = urows.at[s]
        vr = vrows.at[s]

        def group_body(g, carry):
            def edge_body(j, merged):
                e = g * L + j
                acc_bf = None
                for b in range(NB):
                    u = plsc.bitcast(ur[e, pl.ds(b * L, L)], jnp.bfloat16)
                    v = plsc.bitcast(vr[e, pl.ds(b * L, L)], jnp.bfloat16)
                    h = jnp.maximum(u + v, jnp.bfloat16(0.0))
                    p = h * w2regs[b]
                    acc_bf = p if acc_bf is None else acc_bf + p
                lo, hi = plsc.unpack(acc_bf, format=plsc.PackFormat.INTERLEAVED)
                acc = lo + hi
                return jnp.where(lane == j, jnp.sum(acc), merged)

            merged = lax.fori_loop(0, L, edge_body,
                                   jnp.zeros((L,), jnp.float32), unroll=8)
            z = merged + b2reg
            logv[pl.ds(g * L, L)] = 1.0 / (1.0 + jnp.exp(-z))
            return carry

        lax.fori_loop(0, CHUNK // L, group_body, 0, unroll=False)
        pltpu.sync_copy(logv, out_hbm.at[pl.ds(base + cc * CHUNK, CHUNK)])

    issue(0, 0)

    @pl.loop(0, NCHUNK - 1, step=2)
    def _chunk_pair(c):
        for k in range(2):
            cc = c + k
            issue(cc + 1, 1 - k)
            wait(cc, k)
            compute(cc, k)

    wait(NCHUNK - 1, 0)
    compute(NCHUNK - 1, 0)


def _edge_mask(U, V, ei, w2, b2vec):
    mesh = plsc.VectorSubcoreMesh(core_axis_name="c", subcore_axis_name="s")
    run = functools.partial(
        pl.kernel,
        mesh=mesh,
        out_type=jax.ShapeDtypeStruct((N_EDGES,), jnp.float32),
        compiler_params=pltpu.CompilerParams(needs_layout_passes=False, use_tc_tiling_on_sc=False),
        scratch_types=[
            pltpu.VMEM((NCHUNK, CHUNK), jnp.int32),
            pltpu.VMEM((NCHUNK, CHUNK), jnp.int32),
            pltpu.VMEM((2, CHUNK, DIM // 2), jnp.int32),
            pltpu.VMEM((2, CHUNK, DIM // 2), jnp.int32),
            pltpu.VMEM((CHUNK,), jnp.float32),
            pltpu.VMEM((DIM // 2,), jnp.int32),
            pltpu.VMEM((L,), jnp.float32),
            pltpu.SemaphoreType.DMA((2,)),
        ],
    )(_edge_body)
    return run(U, V, ei, w2, b2vec)


def kernel(x, edge_index, W1, b1, W2, b2):
    ei = edge_index.astype(jnp.int32).reshape(2, NW, NCHUNK, CHUNK)
    U, V = _node_projections(x, W1, b1)
    w2f = W2.reshape(DIM)
    w2 = _pack_pair(w2f[: DIM // 2], w2f[DIM // 2:])
    b2vec = jnp.broadcast_to(b2, (L,)).astype(jnp.float32)
    return _edge_mask(U, V, ei, w2, b2vec)


# speedup vs baseline: 1.2628x; 1.2628x over previous
import functools

import jax
import jax.numpy as jnp
from jax import lax
from jax.experimental import pallas as pl
from jax.experimental.pallas import tpu as pltpu
from jax.experimental.pallas import tpu_sc as plsc

N_NODES = 10000
N_EDGES = 320000
DIM = 128
NC = 2
NS = 16
NW = NC * NS
L = 16
EPW = N_EDGES // NW
CHUNK = 80
NCHUNK = EPW // CHUNK
NF = DIM // L
NB = DIM // (2 * L)


def _pack_pair(a, b):
    ia = lax.bitcast_convert_type(a, jnp.int32)
    ib = lax.bitcast_convert_type(b, jnp.int32)
    lo = lax.shift_right_logical(ia + 0x8000, 16)
    hi = jnp.bitwise_and(ib + 0x8000, jnp.int32(-65536))
    return jnp.bitwise_or(hi, lo)


def _proj_body(x_ref, wal_ref, wah_ref, wbl_ref, wbh_ref, b1l_ref, b1h_ref,
               u_ref, v_ref):
    xv = x_ref[...]
    f32 = jnp.float32
    ul = jnp.dot(xv, wal_ref[...], preferred_element_type=f32)
    uh = jnp.dot(xv, wah_ref[...], preferred_element_type=f32)
    vl = jnp.dot(xv, wbl_ref[...], preferred_element_type=f32) + b1l_ref[...]
    vh = jnp.dot(xv, wbh_ref[...], preferred_element_type=f32) + b1h_ref[...]
    u_ref[...] = _pack_pair(ul, uh)
    v_ref[...] = _pack_pair(vl, vh)


def _node_projections(x, W1, b1):
    H = DIM // 2
    wa = W1[:, :DIM].T
    wb = W1[:, DIM:].T
    return pl.pallas_call(
        _proj_body,
        out_shape=[
            jax.ShapeDtypeStruct((N_NODES, H), jnp.int32),
            jax.ShapeDtypeStruct((N_NODES, H), jnp.int32),
        ],
    )(x, wa[:, :H], wa[:, H:], wb[:, :H], wb[:, H:],
      b1[:H].reshape(1, H), b1[H:].reshape(1, H))


def _edge_body(u_hbm, v_hbm, ei_hbm, w2_hbm, b2_hbm, out_hbm,
               idx_r, idx_c, urows, vrows, logv, w2v, b2v, sem):
    cid = lax.axis_index("c")
    sid = lax.axis_index("s")
    wid = sid * NC + cid
    base = wid * EPW

    pltpu.sync_copy(ei_hbm.at[0, wid], idx_r)
    pltpu.sync_copy(ei_hbm.at[1, wid], idx_c)
    pltpu.sync_copy(w2_hbm, w2v)
    pltpu.sync_copy(b2_hbm, b2v)
    b2reg = b2v[...]
    w2regs = [
        plsc.bitcast(w2v[pl.ds(b * L, L)], jnp.bfloat16) for b in range(NB)
    ]
    lane = lax.iota(jnp.int32, L)

    def issue(cc, s):
        pltpu.async_copy(u_hbm.at[idx_r.at[cc]], urows.at[s], sem.at[s])
        pltpu.async_copy(v_hbm.at[idx_c.at[cc]], vrows.at[s], sem.at[s])

    def wait(cc, s):
        pltpu.make_async_copy(u_hbm.at[idx_r.at[cc]], urows.at[s], sem.at[s]).wait()
        pltpu.make_async_copy(v_hbm.at[idx_c.at[cc]], vrows.at[s], sem.at[s]).wait()

    def compute(cc, s):
        ur = urows.at[s]
        vr = vrows.at[s]

        def group_body(g, carry):
            def edge_body(j, merged):
                e = g * L + j
                acc_bf = None
                for b in range(NB):
                    u = plsc.bitcast(ur[e, pl.ds(b * L, L)], jnp.bfloat16)
                    v = plsc.bitcast(vr[e, pl.ds(b * L, L)], jnp.bfloat16)
                    h = jnp.maximum(u + v, jnp.bfloat16(0.0))
                    p = h * w2regs[b]
                    acc_bf = p if acc_bf is None else acc_bf + p
                lo, hi = plsc.unpack(acc_bf, format=plsc.PackFormat.INTERLEAVED)
                acc = lo + hi
                return jnp.where(lane == j, jnp.sum(acc), merged)

            merged = lax.fori_loop(0, L, edge_body,
                                   jnp.zeros((L,), jnp.float32), unroll=4)
            z = merged + b2reg
            logv[pl.ds(g * L, L)] = 1.0 / (1.0 + jnp.exp(-z))
            return carry

        lax.fori_loop(0, CHUNK // L, group_body, 0, unroll=False)
        pltpu.sync_copy(logv, out_hbm.at[pl.ds(base + cc * CHUNK, CHUNK)])

    issue(0, 0)
    issue(1, 1)

    @pl.loop(0, NCHUNK - 2, step=3)
    def _chunk_tri(c):
        for k in range(3):
            cc = c + k
            issue(cc + 2, (k + 2) % 3)
            wait(cc, k)
            compute(cc, k)

    wait(NCHUNK - 2, 0)
    compute(NCHUNK - 2, 0)
    wait(NCHUNK - 1, 1)
    compute(NCHUNK - 1, 1)


def _edge_mask(U, V, ei, w2, b2vec):
    mesh = plsc.VectorSubcoreMesh(core_axis_name="c", subcore_axis_name="s")
    run = functools.partial(
        pl.kernel,
        mesh=mesh,
        out_type=jax.ShapeDtypeStruct((N_EDGES,), jnp.float32),
        compiler_params=pltpu.CompilerParams(needs_layout_passes=False, use_tc_tiling_on_sc=False),
        scratch_types=[
            pltpu.VMEM((NCHUNK, CHUNK), jnp.int32),
            pltpu.VMEM((NCHUNK, CHUNK), jnp.int32),
            pltpu.VMEM((3, CHUNK, DIM // 2), jnp.int32),
            pltpu.VMEM((3, CHUNK, DIM // 2), jnp.int32),
            pltpu.VMEM((CHUNK,), jnp.float32),
            pltpu.VMEM((DIM // 2,), jnp.int32),
            pltpu.VMEM((L,), jnp.float32),
            pltpu.SemaphoreType.DMA((3,)),
        ],
    )(_edge_body)
    return run(U, V, ei, w2, b2vec)


def kernel(x, edge_index, W1, b1, W2, b2):
    ei = edge_index.astype(jnp.int32).reshape(2, NW, NCHUNK, CHUNK)
    U, V = _node_projections(x, W1, b1)
    w2f = W2.reshape(DIM)
    w2 = _pack_pair(w2f[: DIM // 2], w2f[DIM // 2:])
    b2vec = jnp.broadcast_to(b2, (L,)).astype(jnp.float32)
    return _edge_mask(U, V, ei, w2, b2vec)


# speedup vs baseline: 1.2648x; 1.0015x over previous
import functools

import jax
import jax.numpy as jnp
from jax import lax
from jax.experimental import pallas as pl
from jax.experimental.pallas import tpu as pltpu
from jax.experimental.pallas import tpu_sc as plsc

N_NODES = 10000
N_EDGES = 320000
DIM = 128
NC = 2
NS = 16
NW = NC * NS
L = 16
EPW = N_EDGES // NW
CHUNK = 80
NCHUNK = EPW // CHUNK
NF = DIM // L
NB = DIM // (2 * L)


def _pack_pair(a, b):
    ia = lax.bitcast_convert_type(a, jnp.int32)
    ib = lax.bitcast_convert_type(b, jnp.int32)
    lo = lax.shift_right_logical(ia + 0x8000, 16)
    hi = jnp.bitwise_and(ib + 0x8000, jnp.int32(-65536))
    return jnp.bitwise_or(hi, lo)


def _proj_body(x_ref, wal_ref, wah_ref, wbl_ref, wbh_ref, b1l_ref, b1h_ref,
               u_ref, v_ref):
    xv = x_ref[...]
    f32 = jnp.float32
    ul = jnp.dot(xv, wal_ref[...], preferred_element_type=f32)
    uh = jnp.dot(xv, wah_ref[...], preferred_element_type=f32)
    vl = jnp.dot(xv, wbl_ref[...], preferred_element_type=f32) + b1l_ref[...]
    vh = jnp.dot(xv, wbh_ref[...], preferred_element_type=f32) + b1h_ref[...]
    u_ref[...] = _pack_pair(ul, uh)
    v_ref[...] = _pack_pair(vl, vh)


def _node_projections(x, W1, b1):
    H = DIM // 2
    wa = W1[:, :DIM].T
    wb = W1[:, DIM:].T
    return pl.pallas_call(
        _proj_body,
        out_shape=[
            jax.ShapeDtypeStruct((N_NODES, H), jnp.int32),
            jax.ShapeDtypeStruct((N_NODES, H), jnp.int32),
        ],
    )(x, wa[:, :H], wa[:, H:], wb[:, :H], wb[:, H:],
      b1[:H].reshape(1, H), b1[H:].reshape(1, H))


def _edge_body(u_hbm, v_hbm, ei_hbm, w2_hbm, b2_hbm, out_hbm,
               idx_r, idx_c, urows, vrows, logv, w2v, b2v, sem):
    cid = lax.axis_index("c")
    sid = lax.axis_index("s")
    wid = sid * NC + cid
    base = wid * EPW

    pltpu.sync_copy(ei_hbm.at[0, wid], idx_r)
    pltpu.sync_copy(ei_hbm.at[1, wid], idx_c)
    pltpu.sync_copy(w2_hbm, w2v)
    pltpu.sync_copy(b2_hbm, b2v)
    b2reg = b2v[...]
    w2regs = [
        plsc.bitcast(w2v[pl.ds(b * L, L)], jnp.bfloat16) for b in range(NB)
    ]
    lane = lax.iota(jnp.int32, L)

    def issue(cc, s):
        pltpu.async_copy(u_hbm.at[idx_r.at[cc]], urows.at[s], sem.at[s])
        pltpu.async_copy(v_hbm.at[idx_c.at[cc]], vrows.at[s], sem.at[s])

    def wait(cc, s):
        pltpu.make_async_copy(u_hbm.at[idx_r.at[cc]], urows.at[s], sem.at[s]).wait()
        pltpu.make_async_copy(v_hbm.at[idx_c.at[cc]], vrows.at[s], sem.at[s]).wait()

    def compute(cc, s):
        ur = urows.at[s]
        vr = vrows.at[s]

        def group_body(g, carry):
            def edge_body(j, merged):
                e = g * L + j
                acc_bf = None
                for b in range(NB):
                    u = plsc.bitcast(ur[e, pl.ds(b * L, L)], jnp.bfloat16)
                    v = plsc.bitcast(vr[e, pl.ds(b * L, L)], jnp.bfloat16)
                    h = jnp.maximum(u + v, jnp.bfloat16(0.0))
                    p = h * w2regs[b]
                    acc_bf = p if acc_bf is None else acc_bf + p
                lo, hi = plsc.unpack(acc_bf, format=plsc.PackFormat.INTERLEAVED)
                acc = lo + hi
                return jnp.where(lane == j, jnp.sum(acc), merged)

            merged = lax.fori_loop(0, L, edge_body,
                                   jnp.zeros((L,), jnp.float32), unroll=4)
            z = merged + b2reg
            logv[pl.ds(g * L, L)] = 1.0 / (1.0 + jnp.exp(-z))
            return carry

        lax.fori_loop(0, CHUNK // L, group_body, 0, unroll=False)
        pltpu.sync_copy(logv, out_hbm.at[pl.ds(base + cc * CHUNK, CHUNK)])

    issue(0, 0)
    issue(1, 1)
    issue(2, 2)

    @pl.loop(0, NCHUNK - 1, step=4)
    def _chunk_quad(c):
        for k in range(4):
            cc = c + k

            @pl.when(cc + 3 < NCHUNK)
            def _prefetch():
                issue(cc + 3, (k + 3) % 4)

            wait(cc, k)
            compute(cc, k)

    wait(NCHUNK - 1, 0)
    compute(NCHUNK - 1, 0)


def _edge_mask(U, V, ei, w2, b2vec):
    mesh = plsc.VectorSubcoreMesh(core_axis_name="c", subcore_axis_name="s")
    run = functools.partial(
        pl.kernel,
        mesh=mesh,
        out_type=jax.ShapeDtypeStruct((N_EDGES,), jnp.float32),
        compiler_params=pltpu.CompilerParams(needs_layout_passes=False, use_tc_tiling_on_sc=False),
        scratch_types=[
            pltpu.VMEM((NCHUNK, CHUNK), jnp.int32),
            pltpu.VMEM((NCHUNK, CHUNK), jnp.int32),
            pltpu.VMEM((4, CHUNK, DIM // 2), jnp.int32),
            pltpu.VMEM((4, CHUNK, DIM // 2), jnp.int32),
            pltpu.VMEM((CHUNK,), jnp.float32),
            pltpu.VMEM((DIM // 2,), jnp.int32),
            pltpu.VMEM((L,), jnp.float32),
            pltpu.SemaphoreType.DMA((4,)),
        ],
    )(_edge_body)
    return run(U, V, ei, w2, b2vec)


def kernel(x, edge_index, W1, b1, W2, b2):
    ei = edge_index.astype(jnp.int32).reshape(2, NW, NCHUNK, CHUNK)
    U, V = _node_projections(x, W1, b1)
    w2f = W2.reshape(DIM)
    w2 = _pack_pair(w2f[: DIM // 2], w2f[DIM // 2:])
    b2vec = jnp.broadcast_to(b2, (L,)).astype(jnp.float32)
    return _edge_mask(U, V, ei, w2, b2vec)
